# Initial kernel scaffold; baseline (speedup 1.0000x reference)
import jax
import jax.numpy as jnp
from jax.experimental import pallas as pl


def kernel(x, xs, ffn_key, ffn_value, table, state0):
    raise NotImplementedError("write your pallas kernel here")


# speedup vs baseline: 334.6199x; 334.6199x over previous
import functools

import jax
import jax.numpy as jnp
from jax import lax
from jax.experimental import pallas as pl
from jax.experimental.pallas import tpu as pltpu
from jax.experimental.pallas import tpu_sc as plsc

N_BIT = 8
FFN_V = 4
H = 4
DEPTH = 8
T = 2 ** N_BIT
NV = 2 ** DEPTH
NK = NV - 1
B, L, D = 2, 2048, 8
N = B * L
M = N * H
Q2 = 2 * D
RA = 512
CH = 512
_PERM = tuple(range(0, 16, 2)) + tuple(range(1, 16, 2))
_LVL_OFF = [4 * ((1 << d) - 1) for d in range(DEPTH)]


def _lse_neg(a, axis):
    na = -a
    m = jnp.max(na, axis=axis, keepdims=True)
    out = jnp.log(jnp.sum(jnp.exp(na - m), axis=axis, keepdims=True)) + m
    return -out


def _gather_exact(planes, onehot):
    bits = None
    for k in range(4):
        p = jax.lax.dot_general(planes[16 * k:16 * k + 16, :], onehot,
                                (((1,), (0,)), ((), ())),
                                preferred_element_type=jnp.float32)
        pi = p.astype(jnp.int32) << (8 * k)
        bits = pi if bits is None else bits | pi
    return jax.lax.bitcast_convert_type(bits, jnp.float32)


def _tree_body(qb_ref, qs_ref, uk_ref, ul_ref, uv_ref, klvl_ref, vt_ref,
               vsp_ref, o_ref):
    i = pl.program_id(0)
    qf = qb_ref[...]
    qse = (2.0 * qf - 1.0) * qs_ref[...]
    gcol = jax.lax.broadcasted_iota(jnp.int32, (1, RA), 1) + i * RA
    h = gcol % H
    ix = jnp.zeros((1, RA), jnp.int32)
    support = None
    for d in range(DEPTH):
        nd = 1 << d
        node = h * nd + ix
        rows = jax.lax.broadcasted_iota(jnp.int32, (H * nd, RA), 0)
        onehot = (rows == node).astype(jnp.float32)
        kd = klvl_ref[:, _LVL_OFF[d]:_LVL_OFF[d] + H * nd]
        g = _gather_exact(kd, onehot)
        u = uk_ref[16 * d:16 * d + 16, :]
        kb = (u < g).astype(jnp.float32)
        kb0, kb1 = kb[0:8, :], kb[8:16, :]
        key1 = -jnp.logaddexp(-kb0, qse)
        key2 = -jnp.logaddexp(-kb1, -qse)
        lor_s = jnp.logaddexp(key1, key2)
        lor_s = _lse_neg(lor_s, 0)
        lor = ul_ref[d:d + 1, :] < jax.nn.sigmoid(lor_s)
        lorf = lor.astype(jnp.float32)
        lor_s = (2.0 * lorf - 1.0) * lor_s
        ix = 2 * ix + lor.astype(jnp.int32)
        support = lor_s if support is None else -jnp.logaddexp(-support, -lor_s)
    node = h * NV + ix
    rows = jax.lax.broadcasted_iota(jnp.int32, (H * NV, RA), 0)
    onehot = (rows == node).astype(jnp.float32)
    g8 = jax.lax.dot_general(vt_ref[...], onehot, (((1,), (0,)), ((), ())),
                             preferred_element_type=jnp.float32)
    sv = _gather_exact(vsp_ref[...], onehot)
    vb = (uv_ref[0:FFN_V, :] < sv[0:FFN_V, :]).astype(jnp.float32)
    value_s = (2.0 * vb - 1.0) * g8[0:FFN_V, :]
    value_s = -jnp.logaddexp(-value_s, -support)
    xs2 = _lse_neg(value_s, 0)
    xl = (8.0 * vb[0:1, :] + 4.0 * vb[1:2, :]
          + 2.0 * vb[2:3, :] + 1.0 * vb[3:4, :])
    o_ref[...] = jnp.concatenate(
        [xl, xs2, jnp.zeros((6, RA), jnp.float32)], axis=0)


def _table_body(t_ref, s_ref, u_ref, o_ref):
    tv = t_ref[...]
    sg = s_ref[...]
    outs = []
    for b in range(B):
        tb = (u_ref[8 * b:8 * b + 8, :] < sg).astype(jnp.float32)
        ts = tv * (2.0 * tb - 1.0)
        tl = sum((1 << (N_BIT - 1 - k)) * tb[k:k + 1, :] for k in range(N_BIT))
        outs.append((tl, _lse_neg(ts, 0)))
    rb = tv.shape[1]
    o_ref[...] = jnp.concatenate(
        [outs[0][0], outs[1][0], outs[0][1], outs[1][1],
         jnp.zeros((4, rb), jnp.float32)], axis=0)


def _scan_body(st_ref, tv_ref, xs_ref, ss_ref, ob_ref, oz_ref):
    st = st_ref[...]
    ys = -jnp.logaddexp(-tv_ref[...], -xs_ref[...])
    acc = ys
    k = 1
    while k < L:
        shifted = jnp.concatenate(
            [jnp.full((16, k), 1e30, jnp.float32), acc[:, :L - k]], axis=1)
        acc = -jnp.logaddexp(-acc, -shifted)
        k *= 2
    oz_ref[...] = -jnp.logaddexp(-ss_ref[...], -acc)
    bits = [((st >> (N_BIT - 1 - k)) & 1).astype(jnp.float32)
            for k in range(N_BIT)]
    ob_ref[...] = jnp.concatenate(bits, axis=0)


def _run_traj(tl_flat, ts_flat, xl_lanes, s0_lanes, base_lanes):
    mesh = plsc.VectorSubcoreMesh(core_axis_name="c", subcore_axis_name="s")
    nsteps = L // CH

    @functools.partial(
        pl.kernel, mesh=mesh,
        compiler_params=pltpu.CompilerParams(needs_layout_passes=False,
                                             use_tc_tiling_on_sc=False),
        out_type=[jax.ShapeDtypeStruct((L, 16), jnp.int32),
                  jax.ShapeDtypeStruct((L, 16), jnp.float32)],
        scratch_types=[pltpu.VMEM((B * 64 * T,), jnp.int32),
                       pltpu.VMEM((B * 64 * T,), jnp.float32),
                       pltpu.VMEM((CH, 16), jnp.int32),
                       pltpu.VMEM((CH, 16), jnp.int32),
                       pltpu.VMEM((CH, 16), jnp.float32),
                       pltpu.VMEM((16,), jnp.int32),
                       pltpu.VMEM((16,), jnp.int32)])
    def traj(tl_hbm, ts_hbm, xl_hbm, s0_hbm, base_hbm, st_out, tv_out,
             tl_v, ts_v, xl_v, st_v, tv_v, s0_v, base_v):
        cid = lax.axis_index("c")
        sid = lax.axis_index("s")

        @pl.when(jnp.logical_and(cid == 0, sid == 0))
        def _():
            pltpu.sync_copy(tl_hbm, tl_v)
            pltpu.sync_copy(ts_hbm, ts_v)
            pltpu.sync_copy(s0_hbm, s0_v)
            pltpu.sync_copy(base_hbm, base_v)
            base = base_v[...]
            st0 = s0_v[...]

            def chunk(c, st):
                pltpu.sync_copy(xl_hbm.at[pl.ds(c * CH, CH)], xl_v)

                def body(i, st):
                    idx = xl_v[i] * T + base + st
                    t = plsc.load_gather(ts_v, [idx])
                    ns = plsc.load_gather(tl_v, [idx])
                    st_v[i] = ns
                    tv_v[i] = t
                    return ns

                st = lax.fori_loop(0, CH, body, st)
                pltpu.sync_copy(st_v, st_out.at[pl.ds(c * CH, CH)])
                pltpu.sync_copy(tv_v, tv_out.at[pl.ds(c * CH, CH)])
                return st

            st = st0
            for c in range(nsteps):
                st = chunk(c, st)

    return traj(tl_flat, ts_flat, xl_lanes, s0_lanes, base_lanes)


def _bool2long(xb, n_bit):
    shape = xb.shape
    mul = (2 ** (n_bit - 1 - jnp.arange(n_bit))).astype(jnp.int32)
    xf = xb.reshape(-1, n_bit).astype(jnp.int32)
    return (xf * mul[None, :]).sum(axis=-1).reshape(shape[:-1])


def kernel(x, xs, ffn_key, ffn_value, table, state0):
    rk = jax.random.key(42)
    f32 = jnp.float32

    def unif(tag, shape):
        return jax.random.uniform(jax.random.fold_in(rk, tag), shape, f32)

    qb_t = jnp.repeat(x.reshape(N, D), H, axis=0).T.astype(f32)
    qs_t = jnp.repeat(xs.reshape(N, D), H, axis=0).T
    uk_t = jnp.stack([unif(d, (M, Q2)).T[jnp.array(_PERM), :]
                      for d in range(DEPTH)]).reshape(DEPTH * Q2, M)
    ul_t = jnp.stack([unif(100 + d, (M,)) for d in range(DEPTH)])
    uv = unif(999, (N, H, FFN_V)).reshape(M, FFN_V).T
    uv_t = jnp.concatenate([uv, jnp.ones((4, M), f32)], axis=0)
    def _byte_planes(arr):
        bits = jax.lax.bitcast_convert_type(arr, jnp.int32)
        s = arr.shape[0]
        return jnp.concatenate(
            [jnp.pad(((bits >> (8 * k)) & 0xFF).astype(f32),
                     ((0, 16 - s), (0, 0))) for k in range(4)], axis=0)

    sk = jax.nn.sigmoid(ffn_key).reshape(H * NK, Q2)[:, jnp.array(_PERM)]
    klvl = jnp.concatenate(
        [jnp.concatenate([sk[hh * NK + (1 << d) - 1:hh * NK + (2 << d) - 1]
                          for hh in range(H)], axis=0).T
         for d in range(DEPTH)], axis=1)
    klvl = jnp.pad(klvl, ((0, 0), (0, 1024 - klvl.shape[1])))
    klvlp = _byte_planes(klvl)
    vflat = ffn_value.reshape(H * NV, FFN_V)
    vt = jnp.concatenate([vflat.T, jax.nn.sigmoid(vflat).T], axis=0)
    vsp = _byte_planes(jax.nn.sigmoid(vflat).T)

    outa = pl.pallas_call(
        _tree_body,
        grid=(M // RA,),
        in_specs=[pl.BlockSpec((8, RA), lambda i: (0, i)),
                  pl.BlockSpec((8, RA), lambda i: (0, i)),
                  pl.BlockSpec((DEPTH * Q2, RA), lambda i: (0, i)),
                  pl.BlockSpec((8, RA), lambda i: (0, i)),
                  pl.BlockSpec((8, RA), lambda i: (0, i)),
                  pl.BlockSpec((64, 1024), lambda i: (0, 0)),
                  pl.BlockSpec((8, 1024), lambda i: (0, 0)),
                  pl.BlockSpec((64, 1024), lambda i: (0, 0))],
        out_specs=pl.BlockSpec((8, RA), lambda i: (0, i)),
        out_shape=jax.ShapeDtypeStruct((8, M), f32),
    )(qb_t, qs_t, uk_t, ul_t, uv_t, klvlp, vt, vsp)
    xl = outa[0].astype(jnp.int32)
    xs2 = outa[1]

    tsz = 64 * T
    table_tr = table.reshape(tsz, N_BIT).T
    sig_tr = jax.nn.sigmoid(table_tr)
    utb = unif(5000, (B, 64, T, N_BIT)).reshape(B, tsz, N_BIT)
    utb_t = utb.transpose(0, 2, 1).reshape(B * N_BIT, tsz)
    RB = 2048
    outb = pl.pallas_call(
        _table_body,
        grid=(tsz // RB,),
        in_specs=[pl.BlockSpec((8, RB), lambda i: (0, i)),
                  pl.BlockSpec((8, RB), lambda i: (0, i)),
                  pl.BlockSpec((16, RB), lambda i: (0, i))],
        out_specs=pl.BlockSpec((8, RB), lambda i: (0, i)),
        out_shape=jax.ShapeDtypeStruct((8, tsz), f32),
    )(table_tr, sig_tr, utb_t)
    tl_flat = outb[0:2].reshape(-1).astype(jnp.int32)
    ts_flat = outb[2:4].reshape(-1)

    s0e = jnp.broadcast_to(state0, (B,) + state0.shape[1:])
    s0b = unif(6000, s0e.shape) < jax.nn.sigmoid(s0e)
    s0s_ = state0 * (2.0 * s0b.astype(f32) - 1.0)
    s0 = _bool2long(s0b, N_BIT)
    s0s = -jax.nn.logsumexp(-s0s_, axis=-1)

    xl_lanes = jnp.pad(xl.reshape(B, L, H).transpose(1, 0, 2).reshape(L, B * H),
                       ((0, 0), (0, 8)))
    s0_lanes = jnp.pad(s0.reshape(B * H), (0, 8)).astype(jnp.int32)
    lane = jnp.arange(16, dtype=jnp.int32)
    base_lanes = ((lane // H) % B) * (64 * T) + (lane % H) * (16 * T)
    st_flat, tv_flat = _run_traj(tl_flat, ts_flat, xl_lanes, s0_lanes,
                                 base_lanes)

    st_t = st_flat.reshape(L, 16).T
    tv_t = tv_flat.reshape(L, 16).T
    xs2_t = jnp.pad(xs2.reshape(B, L, H).transpose(0, 2, 1).reshape(B * H, L),
                    ((0, 8), (0, 0)))
    ss_t = jnp.broadcast_to(
        jnp.pad(s0s.reshape(B * H), (0, 8))[:, None], (16, L))
    obits, ozs = pl.pallas_call(
        _scan_body,
        in_specs=[pl.BlockSpec((16, L), lambda: (0, 0)),
                  pl.BlockSpec((16, L), lambda: (0, 0)),
                  pl.BlockSpec((16, L), lambda: (0, 0)),
                  pl.BlockSpec((16, L), lambda: (0, 0))],
        out_specs=[pl.BlockSpec((8 * 16, L), lambda: (0, 0)),
                   pl.BlockSpec((16, L), lambda: (0, 0))],
        out_shape=[jax.ShapeDtypeStruct((8 * 16, L), f32),
                   jax.ShapeDtypeStruct((16, L), f32)],
    )(st_t, tv_t, xs2_t, ss_t)

    zb = (obits.reshape(N_BIT, 16, L)[:, :B * H, :]
          .transpose(1, 2, 0)
          .reshape(B, H, L, N_BIT).transpose(0, 2, 1, 3)
          .reshape(B, L, H * N_BIT).astype(bool))
    zs = (ozs[:B * H].reshape(B, H, L).transpose(0, 2, 1)[..., None]
          * jnp.ones((1, 1, 1, N_BIT), f32)).reshape(B, L, H * N_BIT)
    return zb, zs
